# Initial kernel scaffold; baseline (speedup 1.0000x reference)
import jax
import jax.numpy as jnp
from jax.experimental import pallas as pl


def kernel(x, edge_index, batch, W1, b1, W2, b2, W3, b3, linW, linb):
    raise NotImplementedError("write your pallas kernel here")


# speedup vs baseline: 11.4181x; 11.4181x over previous
import functools

import jax
import jax.numpy as jnp
from jax import lax
from jax.experimental import pallas as pl
from jax.experimental.pallas import tpu as pltpu
from jax.experimental.pallas import tpu_sc as plsc

NC = 2
NS = 16
LANES = 16
NW = NC * NS
G = 128

_MESH = dict(core_axis_name="c", subcore_axis_name="s")


def _deg_build(n, e, ch):
    epw = e // NW
    nch = epw // ch
    stripe = (n // NS) // 8 * 8
    tail = n - stripe * NS
    assert epw * NW == e and nch * ch == epw and tail % 8 == 0 and tail >= 0 and stripe % 48 == 0

    @functools.partial(
        pl.kernel,
        out_type=jax.ShapeDtypeStruct((NC, n, LANES), jnp.float32),
        mesh=plsc.VectorSubcoreMesh(**_MESH),
        scratch_types=[
            pltpu.VMEM((ch,), jnp.int32),
            pltpu.VMEM((ch, LANES), jnp.float32),
            pltpu.VMEM((48, LANES), jnp.float32),
            pltpu.VMEM_SHARED((n, LANES), jnp.float32),
        ],
    )
    def deg_kernel(dst_hbm, out_hbm, didx, ones_v, zbuf, deg_sh):
        c = lax.axis_index("c")
        s = lax.axis_index("s")
        z16 = jnp.zeros((LANES,), jnp.float32)
        o16 = jnp.ones((LANES,), jnp.float32)

        @pl.loop(0, ch)
        def _(i):
            ones_v[i, :] = o16

        @pl.loop(0, 48)
        def _(i):
            zbuf[i, :] = z16

        r0 = pl.multiple_of(s * stripe, 8)

        @pl.loop(0, stripe // 48)
        def _(i):
            pltpu.sync_copy(zbuf, deg_sh.at[pl.ds(r0 + i * 48, 48)])

        @pl.when(s == NS - 1)
        def _():
            pltpu.sync_copy(zbuf.at[pl.ds(0, tail)], deg_sh.at[pl.ds(stripe * NS, tail)])

        plsc.subcore_barrier()

        base = (c * NS + s) * epw

        @pl.loop(0, nch)
        def _(i):
            b = pl.multiple_of(base + i * ch, 8)
            pltpu.sync_copy(dst_hbm.at[pl.ds(b, ch)], didx)
            pltpu.sync_copy(ones_v, deg_sh.at[didx], add=True)

        plsc.subcore_barrier()
        pltpu.sync_copy(deg_sh.at[pl.ds(r0, stripe)], out_hbm.at[c, pl.ds(r0, stripe)])

        @pl.when(s == NS - 1)
        def _():
            pltpu.sync_copy(
                deg_sh.at[pl.ds(stripe * NS, tail)],
                out_hbm.at[c, pl.ds(stripe * NS, tail)],
            )

    return deg_kernel


def _edge_build(n, d, e, ch):
    epw = e // NW
    nch = epw // ch
    stripe = (n // NS) // 8 * 8
    tail = n - stripe * NS
    assert epw * NW == e and nch * ch == epw and tail % 8 == 0 and tail >= 0 and stripe % 48 == 0

    @functools.partial(
        pl.kernel,
        out_type=jax.ShapeDtypeStruct((NC, n, d), jnp.float32),
        mesh=plsc.VectorSubcoreMesh(**_MESH),
        scratch_types=[
            pltpu.VMEM((ch,), jnp.int32),
            pltpu.VMEM((ch,), jnp.int32),
            pltpu.VMEM((ch, d), jnp.float32),
            pltpu.VMEM((48, d), jnp.float32),
            pltpu.VMEM_SHARED((n, d), jnp.float32),
            pltpu.SemaphoreType.DMA,
        ],
    )
    def edge_kernel(hp_hbm, src_hbm, dst_hbm, out_hbm, sidx, didx, rows, zbuf, agg_sh, sem):
        c = lax.axis_index("c")
        s = lax.axis_index("s")
        z16 = jnp.zeros((LANES,), jnp.float32)

        @pl.loop(0, 48)
        def _(i):
            @pl.loop(0, d // LANES)
            def _(j):
                zbuf[i, pl.ds(j * LANES, LANES)] = z16

        r0 = pl.multiple_of(s * stripe, 8)

        @pl.loop(0, stripe // 48)
        def _(i):
            pltpu.sync_copy(zbuf, agg_sh.at[pl.ds(r0 + i * 48, 48)])

        @pl.when(s == NS - 1)
        def _():
            pltpu.sync_copy(zbuf.at[pl.ds(0, tail)], agg_sh.at[pl.ds(stripe * NS, tail)])

        plsc.subcore_barrier()

        base = (c * NS + s) * epw

        @pl.loop(0, nch)
        def _(i):
            b = pl.multiple_of(base + i * ch, 8)
            pltpu.sync_copy(src_hbm.at[pl.ds(b, ch)], sidx)
            pltpu.sync_copy(dst_hbm.at[pl.ds(b, ch)], didx)
            pltpu.async_copy(hp_hbm.at[sidx], rows, sem).wait()
            pltpu.sync_copy(rows, agg_sh.at[didx], add=True)

        plsc.subcore_barrier()
        pltpu.sync_copy(agg_sh.at[pl.ds(r0, stripe)], out_hbm.at[c, pl.ds(r0, stripe)])

        @pl.when(s == NS - 1)
        def _():
            pltpu.sync_copy(
                agg_sh.at[pl.ds(stripe * NS, tail)],
                out_hbm.at[c, pl.ds(stripe * NS, tail)],
            )

    return edge_kernel


def _tc_first(x, w, degs):
    n, d = x.shape

    def body(x_ref, w_ref, deg_ref, hp_ref, dinv_ref):
        deg = deg_ref[0, :, 0:1] + deg_ref[1, :, 0:1] + 1.0
        dinv = lax.rsqrt(deg)
        hw = jnp.dot(x_ref[...], w_ref[...], preferred_element_type=jnp.float32)
        hp_ref[...] = hw * dinv
        dinv_ref[...] = dinv

    return pl.pallas_call(
        body,
        out_shape=(
            jax.ShapeDtypeStruct((n, d), jnp.float32),
            jax.ShapeDtypeStruct((n, 1), jnp.float32),
        ),
    )(x, w, degs)


def _tc_mid(agg, hp, dinv, b, w):
    n, d = hp.shape

    def body(agg_ref, hp_ref, dinv_ref, b_ref, w_ref, out_ref):
        z = (agg_ref[0] + agg_ref[1] + hp_ref[...]) * dinv_ref[...] + b_ref[...]
        z = jnp.maximum(z, 0.0)
        out_ref[...] = (
            jnp.dot(z, w_ref[...], preferred_element_type=jnp.float32) * dinv_ref[...]
        )

    return pl.pallas_call(
        body, out_shape=jax.ShapeDtypeStruct((n, d), jnp.float32)
    )(agg, hp, dinv, b, w)


def _tc_final(agg, hp, dinv, b, batch2d, lw, lb):
    n, d = hp.shape

    def body(agg_ref, hp_ref, dinv_ref, b_ref, batch_ref, lw_ref, lb_ref, out_ref):
        t = (agg_ref[0] + agg_ref[1] + hp_ref[...]) * dinv_ref[...] + b_ref[...]
        gid = lax.broadcasted_iota(jnp.int32, (1, G), 1)
        onehot = (batch_ref[...] == gid).astype(jnp.float32)
        psum = lax.dot_general(
            onehot, t, (((0,), (0,)), ((), ())), preferred_element_type=jnp.float32
        )
        cnt = lax.dot_general(
            onehot,
            jnp.ones((n, 1), jnp.float32),
            (((0,), (0,)), ((), ())),
            preferred_element_type=jnp.float32,
        )
        pooled = psum / jnp.maximum(cnt, 1.0)
        out_ref[...] = (
            jnp.dot(pooled, lw_ref[...], preferred_element_type=jnp.float32)
            + lb_ref[...]
        )

    return pl.pallas_call(
        body, out_shape=jax.ShapeDtypeStruct((G, d), jnp.float32)
    )(agg, hp, dinv, b, batch2d, lw, lb)


@functools.lru_cache(maxsize=None)
def _sc_kernels(n, d, e):
    ch = 80
    return _deg_build(n, e, ch), _edge_build(n, d, e, ch)


def kernel(x, edge_index, batch, W1, b1, W2, b2, W3, b3, linW, linb):
    n, d = x.shape
    e = edge_index.shape[1]
    deg_k, edge_k = _sc_kernels(n, d, e)

    src = edge_index[0].astype(jnp.int32)
    dst = edge_index[1].astype(jnp.int32)

    degs = deg_k(dst)
    hp1, dinv = _tc_first(x, W1, degs)
    agg1 = edge_k(hp1, src, dst)
    hp2 = _tc_mid(agg1, hp1, dinv, b1.reshape(1, d), W2)
    agg2 = edge_k(hp2, src, dst)
    hp3 = _tc_mid(agg2, hp2, dinv, b2.reshape(1, d), W3)
    agg3 = edge_k(hp3, src, dst)
    return _tc_final(
        agg3, hp3, dinv, b3.reshape(1, d), batch.reshape(n, 1).astype(jnp.int32),
        linW, linb.reshape(1, d),
    )


# speedup vs baseline: 21.5703x; 1.8891x over previous
import functools

import jax
import jax.numpy as jnp
from jax import lax
from jax.experimental import pallas as pl
from jax.experimental.pallas import tpu as pltpu
from jax.experimental.pallas import tpu_sc as plsc

NC = 2
NS = 16
LANES = 16
NW = NC * NS
G = 128

_MESH = dict(core_axis_name="c", subcore_axis_name="s")


def _deg_build(n, e, ch):
    epw = e // NW
    nch = epw // ch
    stripe = (n // NS) // 8 * 8
    tail = n - stripe * NS
    assert epw * NW == e and nch * ch == epw and 0 <= tail <= 16 and tail % 8 == 0 and stripe % 16 == 0

    @functools.partial(
        pl.kernel,
        out_type=jax.ShapeDtypeStruct((NC, n, LANES), jnp.float32),
        mesh=plsc.VectorSubcoreMesh(**_MESH),
        compiler_params=pltpu.CompilerParams(use_tc_tiling_on_sc=False),
        scratch_types=[
            pltpu.VMEM((nch, ch), jnp.int32),
            pltpu.VMEM((ch, LANES), jnp.float32),
            pltpu.VMEM((16, LANES), jnp.float32),
            pltpu.VMEM_SHARED((n, LANES), jnp.float32),
        ],
    )
    def deg_kernel(dst_hbm, out_hbm, didx_all, ones_v, zbuf, deg_sh):
        c = lax.axis_index("c")
        s = lax.axis_index("s")
        w = c * NS + s
        z16 = jnp.zeros((LANES,), jnp.float32)
        o16 = jnp.ones((LANES,), jnp.float32)

        pltpu.sync_copy(dst_hbm.at[w], didx_all)

        @pl.loop(0, ch)
        def _(i):
            ones_v[i, :] = o16

        @pl.loop(0, 16)
        def _(i):
            zbuf[i, :] = z16

        r0 = pl.multiple_of(s * stripe, 8)

        @pl.loop(0, stripe // 16)
        def _(i):
            pltpu.sync_copy(zbuf, deg_sh.at[pl.ds(r0 + i * 16, 16)])

        @pl.when(s == NS - 1)
        def _():
            pltpu.sync_copy(zbuf.at[pl.ds(0, tail)], deg_sh.at[pl.ds(stripe * NS, tail)])

        plsc.subcore_barrier()

        @pl.loop(0, nch)
        def _(i):
            pltpu.sync_copy(ones_v, deg_sh.at[didx_all.at[i]], add=True)

        plsc.subcore_barrier()
        pltpu.sync_copy(deg_sh.at[pl.ds(r0, stripe)], out_hbm.at[c, pl.ds(r0, stripe)])

        @pl.when(s == NS - 1)
        def _():
            pltpu.sync_copy(
                deg_sh.at[pl.ds(stripe * NS, tail)],
                out_hbm.at[c, pl.ds(stripe * NS, tail)],
            )

    return deg_kernel


def _edge_build(n, d, e, ch):
    epw = e // NW
    nch = epw // ch
    stripe = (n // NS) // 8 * 8
    tail = n - stripe * NS
    assert epw * NW == e and nch * ch == epw and 0 <= tail <= 16 and tail % 8 == 0 and stripe % 16 == 0

    @functools.partial(
        pl.kernel,
        out_type=jax.ShapeDtypeStruct((NC, n, d), jnp.float32),
        mesh=plsc.VectorSubcoreMesh(**_MESH),
        compiler_params=pltpu.CompilerParams(use_tc_tiling_on_sc=False),
        scratch_types=[
            pltpu.VMEM((nch, ch), jnp.int32),
            pltpu.VMEM((nch, ch), jnp.int32),
            pltpu.VMEM((2, ch, d), jnp.float32),
            pltpu.VMEM((16, d), jnp.float32),
            pltpu.VMEM_SHARED((n, d), jnp.float32),
            pltpu.SemaphoreType.DMA((2,)),
        ],
    )
    def edge_kernel(hp_hbm, src_hbm, dst_hbm, out_hbm, sidx_all, didx_all, rows, zbuf, agg_sh, sem):
        c = lax.axis_index("c")
        s = lax.axis_index("s")
        w = c * NS + s
        z16 = jnp.zeros((LANES,), jnp.float32)

        pltpu.sync_copy(src_hbm.at[w], sidx_all)
        pltpu.sync_copy(dst_hbm.at[w], didx_all)

        @pl.loop(0, 16)
        def _(i):
            @pl.loop(0, d // LANES)
            def _(j):
                zbuf[i, pl.ds(j * LANES, LANES)] = z16

        r0 = pl.multiple_of(s * stripe, 8)

        @pl.loop(0, stripe // 16)
        def _(i):
            pltpu.sync_copy(zbuf, agg_sh.at[pl.ds(r0 + i * 16, 16)])

        @pl.when(s == NS - 1)
        def _():
            pltpu.sync_copy(zbuf.at[pl.ds(0, tail)], agg_sh.at[pl.ds(stripe * NS, tail)])

        plsc.subcore_barrier()

        pltpu.async_copy(hp_hbm.at[sidx_all.at[0]], rows.at[0], sem.at[0])

        @pl.loop(0, nch)
        def _(i):
            b = lax.rem(i, 2)
            pltpu.make_async_copy(hp_hbm.at[sidx_all.at[i]], rows.at[b], sem.at[b]).wait()

            @pl.when(i + 1 < nch)
            def _():
                pltpu.async_copy(
                    hp_hbm.at[sidx_all.at[i + 1]], rows.at[1 - b], sem.at[1 - b]
                )

            pltpu.sync_copy(rows.at[b], agg_sh.at[didx_all.at[i]], add=True)

        plsc.subcore_barrier()
        pltpu.sync_copy(agg_sh.at[pl.ds(r0, stripe)], out_hbm.at[c, pl.ds(r0, stripe)])

        @pl.when(s == NS - 1)
        def _():
            pltpu.sync_copy(
                agg_sh.at[pl.ds(stripe * NS, tail)],
                out_hbm.at[c, pl.ds(stripe * NS, tail)],
            )

    return edge_kernel


def _tc_first(x, w, degs):
    n, d = x.shape

    def body(x_ref, w_ref, deg_ref, hp_ref, dinv_ref):
        deg = deg_ref[0, :, 0:1] + deg_ref[1, :, 0:1] + 1.0
        dinv = lax.rsqrt(deg)
        hw = jnp.dot(x_ref[...], w_ref[...], preferred_element_type=jnp.float32)
        hp_ref[...] = hw * dinv
        dinv_ref[...] = dinv

    return pl.pallas_call(
        body,
        out_shape=(
            jax.ShapeDtypeStruct((n, d), jnp.float32),
            jax.ShapeDtypeStruct((n, 1), jnp.float32),
        ),
    )(x, w, degs)


def _tc_mid(agg, hp, dinv, b, w):
    n, d = hp.shape

    def body(agg_ref, hp_ref, dinv_ref, b_ref, w_ref, out_ref):
        z = (agg_ref[0] + agg_ref[1] + hp_ref[...]) * dinv_ref[...] + b_ref[...]
        z = jnp.maximum(z, 0.0)
        out_ref[...] = (
            jnp.dot(z, w_ref[...], preferred_element_type=jnp.float32) * dinv_ref[...]
        )

    return pl.pallas_call(
        body, out_shape=jax.ShapeDtypeStruct((n, d), jnp.float32)
    )(agg, hp, dinv, b, w)


def _tc_final(agg, hp, dinv, b, batch2d, lw, lb):
    n, d = hp.shape

    def body(agg_ref, hp_ref, dinv_ref, b_ref, batch_ref, lw_ref, lb_ref, out_ref):
        t = (agg_ref[0] + agg_ref[1] + hp_ref[...]) * dinv_ref[...] + b_ref[...]
        gid = lax.broadcasted_iota(jnp.int32, (1, G), 1)
        onehot = (batch_ref[...] == gid).astype(jnp.float32)
        psum = lax.dot_general(
            onehot, t, (((0,), (0,)), ((), ())), preferred_element_type=jnp.float32
        )
        cnt = lax.dot_general(
            onehot,
            jnp.ones((n, 1), jnp.float32),
            (((0,), (0,)), ((), ())),
            preferred_element_type=jnp.float32,
        )
        pooled = psum / jnp.maximum(cnt, 1.0)
        out_ref[...] = (
            jnp.dot(pooled, lw_ref[...], preferred_element_type=jnp.float32)
            + lb_ref[...]
        )

    return pl.pallas_call(
        body, out_shape=jax.ShapeDtypeStruct((G, d), jnp.float32)
    )(agg, hp, dinv, b, batch2d, lw, lb)


@functools.lru_cache(maxsize=None)
def _sc_kernels(n, d, e):
    ch = 80
    return _deg_build(n, e, ch), _edge_build(n, d, e, ch)


def kernel(x, edge_index, batch, W1, b1, W2, b2, W3, b3, linW, linb):
    n, d = x.shape
    e = edge_index.shape[1]
    ch = 80
    nch = e // NW // ch
    deg_k, edge_k = _sc_kernels(n, d, e)

    src = edge_index[0].astype(jnp.int32).reshape(NW, nch, ch)
    dst = edge_index[1].astype(jnp.int32).reshape(NW, nch, ch)

    degs = deg_k(dst)
    hp1, dinv = _tc_first(x, W1, degs)
    agg1 = edge_k(hp1, src, dst)
    hp2 = _tc_mid(agg1, hp1, dinv, b1.reshape(1, d), W2)
    agg2 = edge_k(hp2, src, dst)
    hp3 = _tc_mid(agg2, hp2, dinv, b2.reshape(1, d), W3)
    agg3 = edge_k(hp3, src, dst)
    return _tc_final(
        agg3, hp3, dinv, b3.reshape(1, d), batch.reshape(n, 1).astype(jnp.int32),
        linW, linb.reshape(1, d),
    )


# speedup vs baseline: 30.8329x; 1.4294x over previous
import functools

import jax
import jax.numpy as jnp
from jax import lax
from jax.experimental import pallas as pl
from jax.experimental.pallas import tpu as pltpu
from jax.experimental.pallas import tpu_sc as plsc

NC = 2
NS = 16
LANES = 16
NW = NC * NS
G = 128

_MESH = dict(core_axis_name="c", subcore_axis_name="s")


def _deg_build(n, e, ch):
    epw = e // NW
    nch = epw // ch
    stripe = (n // NS) // 8 * 8
    tail = n - stripe * NS
    assert epw * NW == e and nch * ch == epw and 0 <= tail <= 16 and tail % 8 == 0 and stripe % 16 == 0

    @functools.partial(
        pl.kernel,
        out_type=jax.ShapeDtypeStruct((NC, n, LANES), jnp.float32),
        mesh=plsc.VectorSubcoreMesh(**_MESH),
        compiler_params=pltpu.CompilerParams(use_tc_tiling_on_sc=False),
        scratch_types=[
            pltpu.VMEM((nch, ch), jnp.int32),
            pltpu.VMEM((ch, LANES), jnp.float32),
            pltpu.VMEM((16, LANES), jnp.float32),
            pltpu.VMEM_SHARED((n, LANES), jnp.float32),
            pltpu.SemaphoreType.DMA,
        ],
    )
    def deg_kernel(dst_hbm, out_hbm, didx_all, ones_v, zbuf, deg_sh, sem_d):
        c = lax.axis_index("c")
        s = lax.axis_index("s")
        w = c * NS + s
        z16 = jnp.zeros((LANES,), jnp.float32)
        o16 = jnp.ones((LANES,), jnp.float32)

        pltpu.sync_copy(dst_hbm.at[w], didx_all)

        @pl.loop(0, ch)
        def _(i):
            ones_v[i, :] = o16

        @pl.loop(0, 16)
        def _(i):
            zbuf[i, :] = z16

        r0 = pl.multiple_of(s * stripe, 8)

        @pl.loop(0, stripe // 16)
        def _(i):
            pltpu.sync_copy(zbuf, deg_sh.at[pl.ds(r0 + i * 16, 16)])

        @pl.when(s == NS - 1)
        def _():
            pltpu.sync_copy(zbuf.at[pl.ds(0, tail)], deg_sh.at[pl.ds(stripe * NS, tail)])

        plsc.subcore_barrier()

        @pl.loop(0, nch)
        def _(i):
            pltpu.async_copy(ones_v, deg_sh.at[didx_all.at[i]], sem_d, add=True)

            @pl.when(i >= 4)
            def _():
                pltpu.make_async_copy(ones_v, deg_sh.at[didx_all.at[0]], sem_d).wait()

        for _ in range(4):
            pltpu.make_async_copy(ones_v, deg_sh.at[didx_all.at[0]], sem_d).wait()

        plsc.subcore_barrier()
        pltpu.sync_copy(deg_sh.at[pl.ds(r0, stripe)], out_hbm.at[c, pl.ds(r0, stripe)])

        @pl.when(s == NS - 1)
        def _():
            pltpu.sync_copy(
                deg_sh.at[pl.ds(stripe * NS, tail)],
                out_hbm.at[c, pl.ds(stripe * NS, tail)],
            )

    return deg_kernel


def _edge_build(n, d, e, ch):
    epw = e // NW
    nch = epw // ch
    stripe = (n // NS) // 8 * 8
    tail = n - stripe * NS
    assert epw * NW == e and nch * ch == epw and 0 <= tail <= 16 and tail % 8 == 0 and stripe % 16 == 0

    @functools.partial(
        pl.kernel,
        out_type=jax.ShapeDtypeStruct((NC, n, d), jnp.float32),
        mesh=plsc.VectorSubcoreMesh(**_MESH),
        compiler_params=pltpu.CompilerParams(use_tc_tiling_on_sc=False),
        scratch_types=[
            pltpu.VMEM((nch, ch), jnp.int32),
            pltpu.VMEM((nch, ch), jnp.int32),
            pltpu.VMEM((3, ch, d), jnp.float32),
            pltpu.VMEM_SHARED((n, d), jnp.float32),
            pltpu.SemaphoreType.DMA((3,)),
            pltpu.SemaphoreType.DMA((3,)),
        ],
    )
    def edge_kernel(
        hp_hbm, src_hbm, dst_hbm, zero_hbm, out_hbm,
        sidx_all, didx_all, rows, agg_sh, sem_g, sem_s,
    ):
        c = lax.axis_index("c")
        s = lax.axis_index("s")
        w = c * NS + s

        pltpu.sync_copy(src_hbm.at[w], sidx_all)
        pltpu.sync_copy(dst_hbm.at[w], didx_all)

        pltpu.async_copy(hp_hbm.at[sidx_all.at[0]], rows.at[0], sem_g.at[0])
        pltpu.async_copy(hp_hbm.at[sidx_all.at[1]], rows.at[1], sem_g.at[1])

        r0 = pl.multiple_of(s * stripe, 8)
        pltpu.sync_copy(zero_hbm.at[pl.ds(r0, stripe)], agg_sh.at[pl.ds(r0, stripe)])

        @pl.when(s == NS - 1)
        def _():
            pltpu.sync_copy(
                zero_hbm.at[pl.ds(0, tail)], agg_sh.at[pl.ds(stripe * NS, tail)]
            )

        plsc.subcore_barrier()

        @pl.loop(0, nch)
        def _(i):
            b = lax.rem(i, 3)
            pltpu.make_async_copy(
                hp_hbm.at[sidx_all.at[i]], rows.at[b], sem_g.at[b]
            ).wait()
            pltpu.async_copy(rows.at[b], agg_sh.at[didx_all.at[i]], sem_s.at[b], add=True)

            b2 = lax.rem(i + 2, 3)

            @pl.when(i + 2 < nch)
            def _():
                @pl.when(i > 0)
                def _():
                    pltpu.make_async_copy(
                        rows.at[b2], agg_sh.at[didx_all.at[i]], sem_s.at[b2]
                    ).wait()

                pltpu.async_copy(
                    hp_hbm.at[sidx_all.at[i + 2]], rows.at[b2], sem_g.at[b2]
                )

        for slot in range(3):
            pltpu.make_async_copy(
                rows.at[slot], agg_sh.at[didx_all.at[0]], sem_s.at[slot]
            ).wait()

        plsc.subcore_barrier()
        pltpu.sync_copy(agg_sh.at[pl.ds(r0, stripe)], out_hbm.at[c, pl.ds(r0, stripe)])

        @pl.when(s == NS - 1)
        def _():
            pltpu.sync_copy(
                agg_sh.at[pl.ds(stripe * NS, tail)],
                out_hbm.at[c, pl.ds(stripe * NS, tail)],
            )

    return edge_kernel


def _tc_first(x, w, degs):
    n, d = x.shape

    def body(x_ref, w_ref, deg_ref, hp_ref, dinv_ref):
        deg = deg_ref[0, :, 0:1] + deg_ref[1, :, 0:1] + 1.0
        dinv = lax.rsqrt(deg)
        hw = jnp.dot(x_ref[...], w_ref[...], preferred_element_type=jnp.float32)
        hp_ref[...] = hw * dinv
        dinv_ref[...] = dinv

    return pl.pallas_call(
        body,
        out_shape=(
            jax.ShapeDtypeStruct((n, d), jnp.float32),
            jax.ShapeDtypeStruct((n, 1), jnp.float32),
        ),
    )(x, w, degs)


def _tc_mid(agg, hp, dinv, b, w):
    n, d = hp.shape

    def body(agg_ref, hp_ref, dinv_ref, b_ref, w_ref, out_ref):
        z = (agg_ref[0] + agg_ref[1] + hp_ref[...]) * dinv_ref[...] + b_ref[...]
        z = jnp.maximum(z, 0.0)
        out_ref[...] = (
            jnp.dot(z, w_ref[...], preferred_element_type=jnp.float32) * dinv_ref[...]
        )

    return pl.pallas_call(
        body, out_shape=jax.ShapeDtypeStruct((n, d), jnp.float32)
    )(agg, hp, dinv, b, w)


def _tc_final(agg, hp, dinv, b, batch2d, lw, lb):
    n, d = hp.shape

    def body(agg_ref, hp_ref, dinv_ref, b_ref, batch_ref, lw_ref, lb_ref, out_ref):
        t = (agg_ref[0] + agg_ref[1] + hp_ref[...]) * dinv_ref[...] + b_ref[...]
        gid = lax.broadcasted_iota(jnp.int32, (1, G), 1)
        onehot = (batch_ref[...] == gid).astype(jnp.float32)
        psum = lax.dot_general(
            onehot, t, (((0,), (0,)), ((), ())), preferred_element_type=jnp.float32
        )
        cnt = lax.dot_general(
            onehot,
            jnp.ones((n, 1), jnp.float32),
            (((0,), (0,)), ((), ())),
            preferred_element_type=jnp.float32,
        )
        pooled = psum / jnp.maximum(cnt, 1.0)
        out_ref[...] = (
            jnp.dot(pooled, lw_ref[...], preferred_element_type=jnp.float32)
            + lb_ref[...]
        )

    return pl.pallas_call(
        body, out_shape=jax.ShapeDtypeStruct((G, d), jnp.float32)
    )(agg, hp, dinv, b, batch2d, lw, lb)


@functools.lru_cache(maxsize=None)
def _sc_kernels(n, d, e):
    ch = 80
    return _deg_build(n, e, ch), _edge_build(n, d, e, ch)


def kernel(x, edge_index, batch, W1, b1, W2, b2, W3, b3, linW, linb):
    n, d = x.shape
    e = edge_index.shape[1]
    ch = 80
    nch = e // NW // ch
    deg_k, edge_k = _sc_kernels(n, d, e)

    src = edge_index[0].astype(jnp.int32).reshape(NW, nch, ch)
    dst = edge_index[1].astype(jnp.int32).reshape(NW, nch, ch)

    zeros_nd = jnp.zeros((n, d), jnp.float32)

    degs = deg_k(dst)
    hp1, dinv = _tc_first(x, W1, degs)
    agg1 = edge_k(hp1, src, dst, zeros_nd)
    hp2 = _tc_mid(agg1, hp1, dinv, b1.reshape(1, d), W2)
    agg2 = edge_k(hp2, src, dst, zeros_nd)
    hp3 = _tc_mid(agg2, hp2, dinv, b2.reshape(1, d), W3)
    agg3 = edge_k(hp3, src, dst, zeros_nd)
    return _tc_final(
        agg3, hp3, dinv, b3.reshape(1, d), batch.reshape(n, 1).astype(jnp.int32),
        linW, linb.reshape(1, d),
    )


# speedup vs baseline: 33.0520x; 1.0720x over previous
import functools

import jax
import jax.numpy as jnp
from jax import lax
from jax.experimental import pallas as pl
from jax.experimental.pallas import tpu as pltpu
from jax.experimental.pallas import tpu_sc as plsc

NC = 2
NS = 16
LANES = 16
NW = NC * NS
G = 128

_MESH = dict(core_axis_name="c", subcore_axis_name="s")


def _deg_build(n, e, ch):
    epw = e // NW
    nch = epw // ch
    stripe = (n // NS) // 8 * 8
    tail = n - stripe * NS
    assert epw * NW == e and nch * ch == epw and 0 <= tail <= 16 and tail % 8 == 0 and stripe % 16 == 0

    @functools.partial(
        pl.kernel,
        out_type=jax.ShapeDtypeStruct((NC, n, LANES), jnp.float32),
        mesh=plsc.VectorSubcoreMesh(**_MESH),
        compiler_params=pltpu.CompilerParams(use_tc_tiling_on_sc=False),
        scratch_types=[
            pltpu.VMEM((nch, ch), jnp.int32),
            pltpu.VMEM((ch, LANES), jnp.float32),
            pltpu.VMEM((16, LANES), jnp.float32),
            pltpu.VMEM_SHARED((n, LANES), jnp.float32),
            pltpu.SemaphoreType.DMA,
        ],
    )
    def deg_kernel(ei_hbm, out_hbm, didx_all, ones_v, zbuf, deg_sh, sem_d):
        c = lax.axis_index("c")
        s = lax.axis_index("s")
        w = c * NS + s
        z16 = jnp.zeros((LANES,), jnp.float32)
        o16 = jnp.ones((LANES,), jnp.float32)

        pltpu.sync_copy(ei_hbm.at[1, w], didx_all)

        @pl.loop(0, ch)
        def _(i):
            ones_v[i, :] = o16

        @pl.loop(0, 16)
        def _(i):
            zbuf[i, :] = z16

        r0 = pl.multiple_of(s * stripe, 8)

        @pl.loop(0, stripe // 16)
        def _(i):
            pltpu.sync_copy(zbuf, deg_sh.at[pl.ds(r0 + i * 16, 16)])

        @pl.when(s == NS - 1)
        def _():
            pltpu.sync_copy(zbuf.at[pl.ds(0, tail)], deg_sh.at[pl.ds(stripe * NS, tail)])

        plsc.subcore_barrier()

        @pl.loop(0, nch)
        def _(i):
            pltpu.async_copy(ones_v, deg_sh.at[didx_all.at[i]], sem_d, add=True)

            @pl.when(i >= 4)
            def _():
                pltpu.make_async_copy(ones_v, deg_sh.at[didx_all.at[0]], sem_d).wait()

        for _ in range(4):
            pltpu.make_async_copy(ones_v, deg_sh.at[didx_all.at[0]], sem_d).wait()

        plsc.subcore_barrier()
        pltpu.sync_copy(deg_sh.at[pl.ds(r0, stripe)], out_hbm.at[c, pl.ds(r0, stripe)])

        @pl.when(s == NS - 1)
        def _():
            pltpu.sync_copy(
                deg_sh.at[pl.ds(stripe * NS, tail)],
                out_hbm.at[c, pl.ds(stripe * NS, tail)],
            )

    return deg_kernel


def _edge_build(n, d, e, ch):
    epw = e // NW
    nch = epw // ch
    stripe = (n // NS) // 8 * 8
    tail = n - stripe * NS
    assert epw * NW == e and nch * ch == epw and 0 <= tail <= 16 and tail % 8 == 0 and stripe % 16 == 0

    @functools.partial(
        pl.kernel,
        out_type=jax.ShapeDtypeStruct((NC, n, d), jnp.float32),
        mesh=plsc.VectorSubcoreMesh(**_MESH),
        compiler_params=pltpu.CompilerParams(use_tc_tiling_on_sc=False),
        scratch_types=[
            pltpu.VMEM((nch, ch), jnp.int32),
            pltpu.VMEM((nch, ch), jnp.int32),
            pltpu.VMEM((3, ch, d), jnp.float32),
            pltpu.VMEM_SHARED((n, d), jnp.float32),
            pltpu.SemaphoreType.DMA((3,)),
            pltpu.SemaphoreType.DMA((3,)),
        ],
    )
    def edge_kernel(
        hp_hbm, ei_hbm, out_hbm,
        sidx_all, didx_all, rows, agg_sh, sem_g, sem_s,
    ):
        c = lax.axis_index("c")
        s = lax.axis_index("s")
        w = c * NS + s
        z16 = jnp.zeros((LANES,), jnp.float32)

        pltpu.sync_copy(ei_hbm.at[0, w], sidx_all)
        pltpu.sync_copy(ei_hbm.at[1, w], didx_all)

        pltpu.async_copy(hp_hbm.at[sidx_all.at[0]], rows.at[0], sem_g.at[0])
        pltpu.async_copy(hp_hbm.at[sidx_all.at[1]], rows.at[1], sem_g.at[1])

        @pl.loop(0, ch)
        def _(i):
            @pl.loop(0, d // LANES)
            def _(j):
                rows[2, i, pl.ds(j * LANES, LANES)] = z16

        r0 = pl.multiple_of(s * stripe, 8)

        @pl.loop(0, stripe // ch)
        def _(i):
            pltpu.sync_copy(rows.at[2], agg_sh.at[pl.ds(r0 + i * ch, ch)])

        rem = stripe - (stripe // ch) * ch
        if rem:
            pltpu.sync_copy(
                rows.at[2].at[pl.ds(0, rem)],
                agg_sh.at[pl.ds(r0 + (stripe // ch) * ch, rem)],
            )

        @pl.when(s == NS - 1)
        def _():
            pltpu.sync_copy(
                rows.at[2].at[pl.ds(0, tail)], agg_sh.at[pl.ds(stripe * NS, tail)]
            )

        plsc.subcore_barrier()

        @pl.loop(0, nch)
        def _(i):
            b = lax.rem(i, 3)
            pltpu.make_async_copy(
                hp_hbm.at[sidx_all.at[i]], rows.at[b], sem_g.at[b]
            ).wait()
            pltpu.async_copy(rows.at[b], agg_sh.at[didx_all.at[i]], sem_s.at[b], add=True)

            b2 = lax.rem(i + 2, 3)

            @pl.when(i + 2 < nch)
            def _():
                @pl.when(i > 0)
                def _():
                    pltpu.make_async_copy(
                        rows.at[b2], agg_sh.at[didx_all.at[i]], sem_s.at[b2]
                    ).wait()

                pltpu.async_copy(
                    hp_hbm.at[sidx_all.at[i + 2]], rows.at[b2], sem_g.at[b2]
                )

        for slot in range(3):
            pltpu.make_async_copy(
                rows.at[slot], agg_sh.at[didx_all.at[0]], sem_s.at[slot]
            ).wait()

        plsc.subcore_barrier()
        pltpu.sync_copy(agg_sh.at[pl.ds(r0, stripe)], out_hbm.at[c, pl.ds(r0, stripe)])

        @pl.when(s == NS - 1)
        def _():
            pltpu.sync_copy(
                agg_sh.at[pl.ds(stripe * NS, tail)],
                out_hbm.at[c, pl.ds(stripe * NS, tail)],
            )

    return edge_kernel


def _tc_first(x, w, degs):
    n, d = x.shape

    def body(x_ref, w_ref, deg_ref, hp_ref, dinv_ref):
        deg = deg_ref[0, :, 0:1] + deg_ref[1, :, 0:1] + 1.0
        dinv = lax.rsqrt(deg)
        hw = jnp.dot(x_ref[...], w_ref[...], preferred_element_type=jnp.float32)
        hp_ref[...] = hw * dinv
        dinv_ref[...] = dinv

    return pl.pallas_call(
        body,
        out_shape=(
            jax.ShapeDtypeStruct((n, d), jnp.float32),
            jax.ShapeDtypeStruct((n, 1), jnp.float32),
        ),
    )(x, w, degs)


def _tc_mid(agg, hp, dinv, b, w):
    n, d = hp.shape

    def body(agg_ref, hp_ref, dinv_ref, b_ref, w_ref, out_ref):
        z = (agg_ref[0] + agg_ref[1] + hp_ref[...]) * dinv_ref[...] + b_ref[...]
        z = jnp.maximum(z, 0.0)
        out_ref[...] = (
            jnp.dot(z, w_ref[...], preferred_element_type=jnp.float32) * dinv_ref[...]
        )

    return pl.pallas_call(
        body, out_shape=jax.ShapeDtypeStruct((n, d), jnp.float32)
    )(agg, hp, dinv, b, w)


def _tc_final(agg, hp, dinv, b, batch2d, lw, lb):
    n, d = hp.shape

    def body(agg_ref, hp_ref, dinv_ref, b_ref, batch_ref, lw_ref, lb_ref, out_ref):
        t = (agg_ref[0] + agg_ref[1] + hp_ref[...]) * dinv_ref[...] + b_ref[...]
        gid = lax.broadcasted_iota(jnp.int32, (G, 1), 0)
        onehot = (batch_ref[...] == gid).astype(jnp.float32)
        psum = lax.dot_general(
            onehot, t, (((1,), (0,)), ((), ())), preferred_element_type=jnp.float32
        )
        cnt = lax.dot_general(
            onehot,
            jnp.ones((n, 1), jnp.float32),
            (((1,), (0,)), ((), ())),
            preferred_element_type=jnp.float32,
        )
        pooled = psum / jnp.maximum(cnt, 1.0)
        out_ref[...] = (
            jnp.dot(pooled, lw_ref[...], preferred_element_type=jnp.float32)
            + lb_ref[...]
        )

    return pl.pallas_call(
        body, out_shape=jax.ShapeDtypeStruct((G, d), jnp.float32)
    )(agg, hp, dinv, b, batch2d, lw, lb)


@functools.lru_cache(maxsize=None)
def _sc_kernels(n, d, e):
    ch = 80
    return _deg_build(n, e, ch), _edge_build(n, d, e, ch)


def kernel(x, edge_index, batch, W1, b1, W2, b2, W3, b3, linW, linb):
    n, d = x.shape
    e = edge_index.shape[1]
    ch = 80
    nch = e // NW // ch
    deg_k, edge_k = _sc_kernels(n, d, e)

    ei4 = edge_index.astype(jnp.int32).reshape(2, NW, nch, ch)

    degs = deg_k(ei4)
    hp1, dinv = _tc_first(x, W1, degs)
    agg1 = edge_k(hp1, ei4)
    hp2 = _tc_mid(agg1, hp1, dinv, b1.reshape(1, d), W2)
    agg2 = edge_k(hp2, ei4)
    hp3 = _tc_mid(agg2, hp2, dinv, b2.reshape(1, d), W3)
    agg3 = edge_k(hp3, ei4)
    return _tc_final(
        agg3, hp3, dinv, b3.reshape(1, d), batch.reshape(1, n).astype(jnp.int32),
        linW, linb.reshape(1, d),
    )


# speedup vs baseline: 33.1171x; 1.0020x over previous
import functools

import jax
import jax.numpy as jnp
from jax import lax
from jax.experimental import pallas as pl
from jax.experimental.pallas import tpu as pltpu
from jax.experimental.pallas import tpu_sc as plsc

NC = 2
NS = 16
LANES = 16
NW = NC * NS
G = 128

_MESH = dict(core_axis_name="c", subcore_axis_name="s")


def _deg_build(n, e, ch):
    epw = e // NW
    nch = epw // ch
    stripe = (n // NS) // 8 * 8
    tail = n - stripe * NS
    assert epw * NW == e and nch * ch == epw and 0 <= tail <= 16 and tail % 8 == 0 and stripe % 16 == 0

    @functools.partial(
        pl.kernel,
        out_type=jax.ShapeDtypeStruct((NC, n, LANES), jnp.float32),
        mesh=plsc.VectorSubcoreMesh(**_MESH),
        compiler_params=pltpu.CompilerParams(use_tc_tiling_on_sc=False),
        scratch_types=[
            pltpu.VMEM((nch, ch), jnp.int32),
            pltpu.VMEM((ch, LANES), jnp.float32),
            pltpu.VMEM((16, LANES), jnp.float32),
            pltpu.VMEM_SHARED((n, LANES), jnp.float32),
            pltpu.SemaphoreType.DMA,
        ],
    )
    def deg_kernel(ei_hbm, out_hbm, didx_all, ones_v, zbuf, deg_sh, sem_d):
        c = lax.axis_index("c")
        s = lax.axis_index("s")
        w = c * NS + s
        z16 = jnp.zeros((LANES,), jnp.float32)
        o16 = jnp.ones((LANES,), jnp.float32)

        pltpu.sync_copy(ei_hbm.at[1, w], didx_all)

        @pl.loop(0, ch)
        def _(i):
            ones_v[i, :] = o16

        @pl.loop(0, 16)
        def _(i):
            zbuf[i, :] = z16

        r0 = pl.multiple_of(s * stripe, 8)

        @pl.loop(0, stripe // 16)
        def _(i):
            pltpu.sync_copy(zbuf, deg_sh.at[pl.ds(r0 + i * 16, 16)])

        @pl.when(s == NS - 1)
        def _():
            pltpu.sync_copy(zbuf.at[pl.ds(0, tail)], deg_sh.at[pl.ds(stripe * NS, tail)])

        plsc.subcore_barrier()

        @pl.loop(0, nch)
        def _(i):
            pltpu.async_copy(ones_v, deg_sh.at[didx_all.at[i]], sem_d, add=True)

            @pl.when(i >= 4)
            def _():
                pltpu.make_async_copy(ones_v, deg_sh.at[didx_all.at[0]], sem_d).wait()

        for _ in range(4):
            pltpu.make_async_copy(ones_v, deg_sh.at[didx_all.at[0]], sem_d).wait()

        plsc.subcore_barrier()
        pltpu.sync_copy(deg_sh.at[pl.ds(r0, stripe)], out_hbm.at[c, pl.ds(r0, stripe)])

        @pl.when(s == NS - 1)
        def _():
            pltpu.sync_copy(
                deg_sh.at[pl.ds(stripe * NS, tail)],
                out_hbm.at[c, pl.ds(stripe * NS, tail)],
            )

    return deg_kernel


def _edge_build(n, d, e, ch):
    epw = e // NW
    nch = epw // ch
    stripe = (n // NS) // 8 * 8
    tail = n - stripe * NS
    assert epw * NW == e and nch * ch == epw and 0 <= tail <= 16 and tail % 8 == 0 and stripe % 16 == 0

    @functools.partial(
        pl.kernel,
        out_type=jax.ShapeDtypeStruct((NC, n, d), jnp.float32),
        mesh=plsc.VectorSubcoreMesh(**_MESH),
        compiler_params=pltpu.CompilerParams(use_tc_tiling_on_sc=False),
        scratch_types=[
            pltpu.VMEM((2, nch, ch), jnp.int32),
            pltpu.VMEM((3, ch, d), jnp.float32),
            pltpu.VMEM_SHARED((n, d), jnp.float32),
            pltpu.SemaphoreType.DMA((3,)),
            pltpu.SemaphoreType.DMA((3,)),
        ],
    )
    def edge_kernel(
        hp_hbm, ei_hbm, out_hbm,
        idx_all, rows, agg_sh, sem_g, sem_s,
    ):
        c = lax.axis_index("c")
        s = lax.axis_index("s")
        w = c * NS + s
        z16 = jnp.zeros((LANES,), jnp.float32)

        pltpu.sync_copy(ei_hbm.at[:, w], idx_all)
        sidx_all = idx_all.at[0]
        didx_all = idx_all.at[1]

        pltpu.async_copy(hp_hbm.at[sidx_all.at[0]], rows.at[0], sem_g.at[0])
        pltpu.async_copy(hp_hbm.at[sidx_all.at[1]], rows.at[1], sem_g.at[1])

        @pl.loop(0, ch)
        def _(i):
            @pl.loop(0, d // LANES)
            def _(j):
                rows[2, i, pl.ds(j * LANES, LANES)] = z16

        r0 = pl.multiple_of(s * stripe, 8)

        @pl.loop(0, stripe // ch)
        def _(i):
            pltpu.sync_copy(rows.at[2], agg_sh.at[pl.ds(r0 + i * ch, ch)])

        rem = stripe - (stripe // ch) * ch
        if rem:
            pltpu.sync_copy(
                rows.at[2].at[pl.ds(0, rem)],
                agg_sh.at[pl.ds(r0 + (stripe // ch) * ch, rem)],
            )

        @pl.when(s == NS - 1)
        def _():
            pltpu.sync_copy(
                rows.at[2].at[pl.ds(0, tail)], agg_sh.at[pl.ds(stripe * NS, tail)]
            )

        plsc.subcore_barrier()

        @pl.loop(0, nch)
        def _(i):
            b = lax.rem(i, 3)
            pltpu.make_async_copy(
                hp_hbm.at[sidx_all.at[i]], rows.at[b], sem_g.at[b]
            ).wait()
            pltpu.async_copy(rows.at[b], agg_sh.at[didx_all.at[i]], sem_s.at[b], add=True)

            b2 = lax.rem(i + 2, 3)

            @pl.when(i + 2 < nch)
            def _():
                @pl.when(i > 0)
                def _():
                    pltpu.make_async_copy(
                        rows.at[b2], agg_sh.at[didx_all.at[i]], sem_s.at[b2]
                    ).wait()

                pltpu.async_copy(
                    hp_hbm.at[sidx_all.at[i + 2]], rows.at[b2], sem_g.at[b2]
                )

        for slot in range(3):
            pltpu.make_async_copy(
                rows.at[slot], agg_sh.at[didx_all.at[0]], sem_s.at[slot]
            ).wait()

        plsc.subcore_barrier()
        pltpu.sync_copy(agg_sh.at[pl.ds(r0, stripe)], out_hbm.at[c, pl.ds(r0, stripe)])

        @pl.when(s == NS - 1)
        def _():
            pltpu.sync_copy(
                agg_sh.at[pl.ds(stripe * NS, tail)],
                out_hbm.at[c, pl.ds(stripe * NS, tail)],
            )

    return edge_kernel


def _tc_matmul(x, w):
    n, d = x.shape

    def body(x_ref, w_ref, out_ref):
        out_ref[...] = jnp.dot(x_ref[...], w_ref[...], preferred_element_type=jnp.float32)

    return pl.pallas_call(
        body, out_shape=jax.ShapeDtypeStruct((n, d), jnp.float32)
    )(x, w)


def _tc_scale(hw, degs):
    n, d = hw.shape

    def body(hw_ref, deg_ref, hp_ref, dinv_ref):
        deg = deg_ref[0, :, 0:1] + deg_ref[1, :, 0:1] + 1.0
        dinv = lax.rsqrt(deg)
        hp_ref[...] = hw_ref[...] * dinv
        dinv_ref[...] = dinv

    return pl.pallas_call(
        body,
        out_shape=(
            jax.ShapeDtypeStruct((n, d), jnp.float32),
            jax.ShapeDtypeStruct((n, 1), jnp.float32),
        ),
    )(hw, degs)


def _tc_mid(agg, hp, dinv, b, w):
    n, d = hp.shape

    def body(agg_ref, hp_ref, dinv_ref, b_ref, w_ref, out_ref):
        z = (agg_ref[0] + agg_ref[1] + hp_ref[...]) * dinv_ref[...] + b_ref[...]
        z = jnp.maximum(z, 0.0)
        out_ref[...] = (
            jnp.dot(z, w_ref[...], preferred_element_type=jnp.float32) * dinv_ref[...]
        )

    return pl.pallas_call(
        body, out_shape=jax.ShapeDtypeStruct((n, d), jnp.float32)
    )(agg, hp, dinv, b, w)


def _tc_final(agg, hp, dinv, b, batch2d, lw, lb):
    n, d = hp.shape

    def body(agg_ref, hp_ref, dinv_ref, b_ref, batch_ref, lw_ref, lb_ref, out_ref):
        t = (agg_ref[0] + agg_ref[1] + hp_ref[...]) * dinv_ref[...] + b_ref[...]
        gid = lax.broadcasted_iota(jnp.int32, (G, 1), 0)
        onehot = (batch_ref[...] == gid).astype(jnp.float32)
        psum = lax.dot_general(
            onehot, t, (((1,), (0,)), ((), ())), preferred_element_type=jnp.float32
        )
        cnt = lax.dot_general(
            onehot,
            jnp.ones((n, 1), jnp.float32),
            (((1,), (0,)), ((), ())),
            preferred_element_type=jnp.float32,
        )
        pooled = psum / jnp.maximum(cnt, 1.0)
        out_ref[...] = (
            jnp.dot(pooled, lw_ref[...], preferred_element_type=jnp.float32)
            + lb_ref[...]
        )

    return pl.pallas_call(
        body, out_shape=jax.ShapeDtypeStruct((G, d), jnp.float32)
    )(agg, hp, dinv, b, batch2d, lw, lb)


@functools.lru_cache(maxsize=None)
def _sc_kernels(n, d, e):
    ch = 80
    return _deg_build(n, e, ch), _edge_build(n, d, e, ch)


def kernel(x, edge_index, batch, W1, b1, W2, b2, W3, b3, linW, linb):
    n, d = x.shape
    e = edge_index.shape[1]
    ch = 80
    nch = e // NW // ch
    deg_k, edge_k = _sc_kernels(n, d, e)

    ei4 = edge_index.astype(jnp.int32).reshape(2, NW, nch, ch)

    degs = deg_k(ei4)
    hw1 = _tc_matmul(x, W1)
    hp1, dinv = _tc_scale(hw1, degs)
    agg1 = edge_k(hp1, ei4)
    hp2 = _tc_mid(agg1, hp1, dinv, b1.reshape(1, d), W2)
    agg2 = edge_k(hp2, ei4)
    hp3 = _tc_mid(agg2, hp2, dinv, b2.reshape(1, d), W3)
    agg3 = edge_k(hp3, ei4)
    return _tc_final(
        agg3, hp3, dinv, b3.reshape(1, d), batch.reshape(1, n).astype(jnp.int32),
        linW, linb.reshape(1, d),
    )


# speedup vs baseline: 33.1634x; 1.0014x over previous
import functools

import jax
import jax.numpy as jnp
from jax import lax
from jax.experimental import pallas as pl
from jax.experimental.pallas import tpu as pltpu
from jax.experimental.pallas import tpu_sc as plsc

NC = 2
NS = 16
LANES = 16
NW = NC * NS
G = 128

_MESH = dict(core_axis_name="c", subcore_axis_name="s")


def _deg_build(n, e, ch):
    epw = e // NW
    nch = epw // ch
    stripe = (n // NS) // 8 * 8
    tail = n - stripe * NS
    assert epw * NW == e and nch * ch == epw and 0 <= tail <= 16 and tail % 8 == 0 and stripe % 16 == 0

    @functools.partial(
        pl.kernel,
        out_type=jax.ShapeDtypeStruct((NC, n, LANES), jnp.float32),
        mesh=plsc.VectorSubcoreMesh(**_MESH),
        compiler_params=pltpu.CompilerParams(use_tc_tiling_on_sc=False),
        scratch_types=[
            pltpu.VMEM((nch, ch), jnp.int32),
            pltpu.VMEM((ch, LANES), jnp.float32),
            pltpu.VMEM((16, LANES), jnp.float32),
            pltpu.VMEM_SHARED((n, LANES), jnp.float32),
            pltpu.SemaphoreType.DMA,
        ],
    )
    def deg_kernel(ei_hbm, out_hbm, didx_all, ones_v, zbuf, deg_sh, sem_d):
        c = lax.axis_index("c")
        s = lax.axis_index("s")
        w = c * NS + s
        z16 = jnp.zeros((LANES,), jnp.float32)
        o16 = jnp.ones((LANES,), jnp.float32)

        pltpu.sync_copy(ei_hbm.at[1, w], didx_all)

        @pl.loop(0, ch)
        def _(i):
            ones_v[i, :] = o16

        @pl.loop(0, 16)
        def _(i):
            zbuf[i, :] = z16

        r0 = pl.multiple_of(s * stripe, 8)

        @pl.loop(0, stripe // 16)
        def _(i):
            pltpu.sync_copy(zbuf, deg_sh.at[pl.ds(r0 + i * 16, 16)])

        @pl.when(s == NS - 1)
        def _():
            pltpu.sync_copy(zbuf.at[pl.ds(0, tail)], deg_sh.at[pl.ds(stripe * NS, tail)])

        plsc.subcore_barrier()

        @pl.loop(0, nch)
        def _(i):
            pltpu.async_copy(ones_v, deg_sh.at[didx_all.at[i]], sem_d, add=True)

            @pl.when(i >= 4)
            def _():
                pltpu.make_async_copy(ones_v, deg_sh.at[didx_all.at[0]], sem_d).wait()

        for _ in range(4):
            pltpu.make_async_copy(ones_v, deg_sh.at[didx_all.at[0]], sem_d).wait()

        plsc.subcore_barrier()
        pltpu.sync_copy(deg_sh.at[pl.ds(r0, stripe)], out_hbm.at[c, pl.ds(r0, stripe)])

        @pl.when(s == NS - 1)
        def _():
            pltpu.sync_copy(
                deg_sh.at[pl.ds(stripe * NS, tail)],
                out_hbm.at[c, pl.ds(stripe * NS, tail)],
            )

    return deg_kernel


def _edge_build(n, d, e, ch):
    epw = e // NW
    nch = epw // ch
    stripe = (n // NS) // 8 * 8
    tail = n - stripe * NS
    assert epw * NW == e and nch * ch == epw and 0 <= tail <= 16 and tail % 8 == 0 and stripe % 16 == 0

    @functools.partial(
        pl.kernel,
        out_type=jax.ShapeDtypeStruct((NC, n, d), jnp.float32),
        mesh=plsc.VectorSubcoreMesh(**_MESH),
        compiler_params=pltpu.CompilerParams(use_tc_tiling_on_sc=False),
        scratch_types=[
            pltpu.VMEM((2, nch, ch), jnp.int32),
            pltpu.VMEM((3, ch, d), jnp.float32),
            pltpu.VMEM_SHARED((n, d), jnp.float32),
            pltpu.SemaphoreType.DMA((3,)),
            pltpu.SemaphoreType.DMA((3,)),
        ],
    )
    def edge_kernel(
        hp_hbm, ei_hbm, out_hbm,
        idx_all, rows, agg_sh, sem_g, sem_s,
    ):
        c = lax.axis_index("c")
        s = lax.axis_index("s")
        w = c * NS + s
        z16 = jnp.zeros((LANES,), jnp.float32)

        pltpu.sync_copy(ei_hbm.at[:, w], idx_all)
        sidx_all = idx_all.at[0]
        didx_all = idx_all.at[1]

        pltpu.async_copy(hp_hbm.at[sidx_all.at[0]], rows.at[0], sem_g.at[0])
        pltpu.async_copy(hp_hbm.at[sidx_all.at[1]], rows.at[1], sem_g.at[1])

        @pl.loop(0, ch)
        def _(i):
            @pl.loop(0, d // LANES)
            def _(j):
                rows[2, i, pl.ds(j * LANES, LANES)] = z16

        r0 = pl.multiple_of(s * stripe, 8)

        @pl.loop(0, stripe // ch)
        def _(i):
            pltpu.sync_copy(rows.at[2], agg_sh.at[pl.ds(r0 + i * ch, ch)])

        rem = stripe - (stripe // ch) * ch
        if rem:
            pltpu.sync_copy(
                rows.at[2].at[pl.ds(0, rem)],
                agg_sh.at[pl.ds(r0 + (stripe // ch) * ch, rem)],
            )

        @pl.when(s == NS - 1)
        def _():
            pltpu.sync_copy(
                rows.at[2].at[pl.ds(0, tail)], agg_sh.at[pl.ds(stripe * NS, tail)]
            )

        plsc.subcore_barrier()

        @pl.loop(0, nch)
        def _(i):
            b = lax.rem(i, 3)
            pltpu.make_async_copy(
                hp_hbm.at[sidx_all.at[i]], rows.at[b], sem_g.at[b]
            ).wait()
            pltpu.async_copy(rows.at[b], agg_sh.at[didx_all.at[i]], sem_s.at[b], add=True)

            b2 = lax.rem(i + 2, 3)

            @pl.when(i + 2 < nch)
            def _():
                @pl.when(i > 0)
                def _():
                    pltpu.make_async_copy(
                        rows.at[b2], agg_sh.at[didx_all.at[i]], sem_s.at[b2]
                    ).wait()

                pltpu.async_copy(
                    hp_hbm.at[sidx_all.at[i + 2]], rows.at[b2], sem_g.at[b2]
                )

        for slot in range(3):
            pltpu.make_async_copy(
                rows.at[slot], agg_sh.at[didx_all.at[0]], sem_s.at[slot]
            ).wait()

        plsc.subcore_barrier()
        pltpu.sync_copy(agg_sh.at[pl.ds(r0, stripe)], out_hbm.at[c, pl.ds(r0, stripe)])

        @pl.when(s == NS - 1)
        def _():
            pltpu.sync_copy(
                agg_sh.at[pl.ds(stripe * NS, tail)],
                out_hbm.at[c, pl.ds(stripe * NS, tail)],
            )

    return edge_kernel


def _tc_matmul(x, w):
    n, d = x.shape

    def body(x_ref, w_ref, out_ref):
        out_ref[...] = jnp.dot(x_ref[...], w_ref[...], preferred_element_type=jnp.float32)

    return pl.pallas_call(
        body, out_shape=jax.ShapeDtypeStruct((n, d), jnp.float32)
    )(x, w)


def _tc_scale(hw, degs):
    n, d = hw.shape
    blk = 2000
    nb = n // blk
    assert nb * blk == n

    def body(hw_ref, deg_ref, hp_ref, dinv_ref):
        deg = deg_ref[0, :, 0:1] + deg_ref[1, :, 0:1] + 1.0
        dinv = lax.rsqrt(deg)
        hp_ref[...] = hw_ref[...] * dinv
        dinv_ref[...] = dinv

    return pl.pallas_call(
        body,
        grid=(nb,),
        in_specs=[
            pl.BlockSpec((blk, d), lambda i: (i, 0)),
            pl.BlockSpec((2, blk, LANES), lambda i: (0, i, 0)),
        ],
        out_specs=(
            pl.BlockSpec((blk, d), lambda i: (i, 0)),
            pl.BlockSpec((blk, 1), lambda i: (i, 0)),
        ),
        out_shape=(
            jax.ShapeDtypeStruct((n, d), jnp.float32),
            jax.ShapeDtypeStruct((n, 1), jnp.float32),
        ),
    )(hw, degs)


def _tc_mid(agg, hp, dinv, b, w):
    n, d = hp.shape
    blk = 2000
    nb = n // blk
    assert nb * blk == n

    def body(agg_ref, hp_ref, dinv_ref, b_ref, w_ref, out_ref):
        z = (agg_ref[0] + agg_ref[1] + hp_ref[...]) * dinv_ref[...] + b_ref[...]
        z = jnp.maximum(z, 0.0)
        out_ref[...] = (
            jnp.dot(z, w_ref[...], preferred_element_type=jnp.float32) * dinv_ref[...]
        )

    return pl.pallas_call(
        body,
        grid=(nb,),
        in_specs=[
            pl.BlockSpec((2, blk, d), lambda i: (0, i, 0)),
            pl.BlockSpec((blk, d), lambda i: (i, 0)),
            pl.BlockSpec((blk, 1), lambda i: (i, 0)),
            pl.BlockSpec((1, d), lambda i: (0, 0)),
            pl.BlockSpec((d, d), lambda i: (0, 0)),
        ],
        out_specs=pl.BlockSpec((blk, d), lambda i: (i, 0)),
        out_shape=jax.ShapeDtypeStruct((n, d), jnp.float32),
    )(agg, hp, dinv, b, w)


def _tc_final(agg, hp, dinv, b, batch2d, lw, lb):
    n, d = hp.shape

    def body(agg_ref, hp_ref, dinv_ref, b_ref, batch_ref, lw_ref, lb_ref, out_ref):
        t = (agg_ref[0] + agg_ref[1] + hp_ref[...]) * dinv_ref[...] + b_ref[...]
        gid = lax.broadcasted_iota(jnp.int32, (G, 1), 0)
        onehot = (batch_ref[...] == gid).astype(jnp.float32)
        psum = lax.dot_general(
            onehot, t, (((1,), (0,)), ((), ())), preferred_element_type=jnp.float32
        )
        cnt = lax.dot_general(
            onehot,
            jnp.ones((n, 1), jnp.float32),
            (((1,), (0,)), ((), ())),
            preferred_element_type=jnp.float32,
        )
        pooled = psum / jnp.maximum(cnt, 1.0)
        out_ref[...] = (
            jnp.dot(pooled, lw_ref[...], preferred_element_type=jnp.float32)
            + lb_ref[...]
        )

    return pl.pallas_call(
        body, out_shape=jax.ShapeDtypeStruct((G, d), jnp.float32)
    )(agg, hp, dinv, b, batch2d, lw, lb)


@functools.lru_cache(maxsize=None)
def _sc_kernels(n, d, e):
    ch = 80
    return _deg_build(n, e, ch), _edge_build(n, d, e, ch)


def kernel(x, edge_index, batch, W1, b1, W2, b2, W3, b3, linW, linb):
    n, d = x.shape
    e = edge_index.shape[1]
    ch = 80
    nch = e // NW // ch
    deg_k, edge_k = _sc_kernels(n, d, e)

    ei4 = edge_index.astype(jnp.int32).reshape(2, NW, nch, ch)

    degs = deg_k(ei4)
    hw1 = _tc_matmul(x, W1)
    hp1, dinv = _tc_scale(hw1, degs)
    agg1 = edge_k(hp1, ei4)
    hp2 = _tc_mid(agg1, hp1, dinv, b1.reshape(1, d), W2)
    agg2 = edge_k(hp2, ei4)
    hp3 = _tc_mid(agg2, hp2, dinv, b2.reshape(1, d), W3)
    agg3 = edge_k(hp3, ei4)
    return _tc_final(
        agg3, hp3, dinv, b3.reshape(1, d), batch.reshape(1, n).astype(jnp.int32),
        linW, linb.reshape(1, d),
    )
